# baseline (speedup 1.0000x reference)
import functools

import jax
import jax.numpy as jnp
from jax import lax
from jax.experimental import pallas as pl
from jax.experimental.pallas import tpu as pltpu
from jax.experimental.pallas import tpu_sc as plsc

_D = 2048
_SEG_FLAT = 4096
_N_FLAT = 16384
_NC = 2
_NS = 16
_NW = _NC * _NS
_ROWS_PER_W = _N_FLAT // _NW
_C = 32
_L = 16


def _sc_add_body(x_hbm, emb_hbm, out_hbm, emb_v, buf):
    wid = lax.axis_index("s") * _NC + lax.axis_index("c")
    base = wid * _ROWS_PER_W
    seg = base // _SEG_FLAT
    pltpu.sync_copy(emb_hbm.at[seg], emb_v)

    def chunk(g, carry):
        row0 = base + g * _C
        pltpu.sync_copy(x_hbm.at[pl.ds(row0, _C)], buf)

        def rowloop(r, c2):
            for k in range(_D // _L):
                sl = pl.ds(k * _L, _L)
                buf[r, sl] = buf[r, sl] + emb_v[sl]
            return c2

        lax.fori_loop(0, _C, rowloop, 0)
        pltpu.sync_copy(buf, out_hbm.at[pl.ds(row0, _C)])
        return carry

    lax.fori_loop(0, _ROWS_PER_W // _C, chunk, 0)


_sc_add = functools.partial(
    pl.kernel,
    mesh=plsc.VectorSubcoreMesh(core_axis_name="c", subcore_axis_name="s"),
    out_type=jax.ShapeDtypeStruct((_N_FLAT, _D), jnp.float32),
    scratch_types=[
        pltpu.VMEM((_D,), jnp.float32),
        pltpu.VMEM((_C, _D), jnp.float32),
    ],
)(_sc_add_body)


def kernel(x, emb):
    n, b, d = x.shape
    out = _sc_add(x.reshape(n * b, d), emb)
    return out.reshape(n, b, d)


# speedup vs baseline: 1.5509x; 1.5509x over previous
import functools

import jax
import jax.numpy as jnp
from jax import lax
from jax.experimental import pallas as pl
from jax.experimental.pallas import tpu as pltpu
from jax.experimental.pallas import tpu_sc as plsc

_D = 2048
_SEG_FLAT = 4096
_N_FLAT = 16384
_NC = 2
_NS = 16
_NW = _NC * _NS
_ROWS_PER_W = _N_FLAT // _NW
_C = 32
_L = 16
_KB = 16


def _sc_add_body(x_hbm, emb_hbm, out_hbm, emb_v, buf):
    wid = lax.axis_index("s") * _NC + lax.axis_index("c")
    base = wid * _ROWS_PER_W
    seg = base // _SEG_FLAT
    pltpu.sync_copy(emb_hbm.at[seg], emb_v)

    def chunk(g, carry):
        row0 = base + g * _C
        pltpu.sync_copy(x_hbm.at[pl.ds(row0, _C)], buf)

        for kb in range(0, _D // _L, _KB):
            ev = [emb_v[pl.ds((kb + j) * _L, _L)] for j in range(_KB)]

            def rowloop(r, c2):
                for j in range(_KB):
                    plsc.addupdate(buf.at[r, pl.ds((kb + j) * _L, _L)], ev[j])
                return c2

            lax.fori_loop(0, _C, rowloop, 0)
        pltpu.sync_copy(buf, out_hbm.at[pl.ds(row0, _C)])
        return carry

    lax.fori_loop(0, _ROWS_PER_W // _C, chunk, 0)


_sc_add = functools.partial(
    pl.kernel,
    mesh=plsc.VectorSubcoreMesh(core_axis_name="c", subcore_axis_name="s"),
    out_type=jax.ShapeDtypeStruct((_N_FLAT, _D), jnp.float32),
    scratch_types=[
        pltpu.VMEM((_D,), jnp.float32),
        pltpu.VMEM((_C, _D), jnp.float32),
    ],
)(_sc_add_body)


def kernel(x, emb):
    n, b, d = x.shape
    out = _sc_add(x.reshape(n * b, d), emb)
    return out.reshape(n, b, d)
